# baseline (speedup 1.0000x reference)
import functools

import jax
import jax.numpy as jnp
from jax import lax
from jax.experimental import pallas as pl
from jax.experimental.pallas import tpu as pltpu
from jax.experimental.pallas import tpu_sc as plsc

E = 8
DIM = 1024
FF = 2048
TOK_BLK = 512
LANES = 128
ROW_BLK = 128
TOTAL = 4096
NSLOT = 2 * TOTAL
NPAD = NSLOT + E * ROW_BLK
NTILES = NPAD // ROW_BLK
NTILES_PAD = 80
NW = 32
CHUNK = NSLOT // NW
N_CHUNKS = TOTAL // (TOK_BLK // 4)


def _router_body(x_ref, wr_ref, br_ref, ti_ref, tw_ref, ch_ref, z_ref,
                 aux_ref, imp_ref, cnt_ref, *, n_tok, n_blocks):
    t = pl.program_id(0)
    x = x_ref[...]
    wr = wr_ref[...]
    logits = jax.lax.dot_general(x, wr, (((1,), (1,)), ((), ())),
                                 preferred_element_type=jnp.float32)
    logits = logits + br_ref[...]
    lane = jax.lax.broadcasted_iota(jnp.int32, logits.shape, 1)
    valid = lane < E
    logits = jnp.where(valid, logits, jnp.float32(-1e30))

    m = jnp.max(logits, axis=1, keepdims=True)
    el = jnp.where(valid, jnp.exp(logits - m), 0.0)
    s = jnp.sum(el, axis=1, keepdims=True)
    probs = el / s
    lse = m + jnp.log(s)

    big = jnp.int32(LANES)
    p1 = jnp.max(probs, axis=1, keepdims=True)
    i1 = jnp.min(jnp.where(probs == p1, lane, big), axis=1, keepdims=True)
    probs2 = jnp.where(lane == i1, -1.0, probs)
    p2 = jnp.max(probs2, axis=1, keepdims=True)
    i2 = jnp.min(jnp.where(probs2 == p2, lane, big), axis=1, keepdims=True)

    denom = p1 + p2 + 1e-12
    ti_ref[...] = jnp.concatenate([i1, i2], axis=1)
    tw_ref[...] = jnp.concatenate([p1 / denom, p2 / denom], axis=1)

    onehot = (jnp.where(lane == i1, 1.0, 0.0) + jnp.where(lane == i2, 1.0, 0.0))
    sub = jnp.sum(onehot.reshape(4, TOK_BLK // 4, LANES), axis=1)
    ch_ref[...] = sub.astype(jnp.int32).reshape(1, 4, LANES)

    @pl.when(t == 0)
    def _():
        imp_ref[...] = jnp.zeros_like(imp_ref)
        cnt_ref[...] = jnp.zeros_like(cnt_ref)
        z_ref[...] = jnp.zeros_like(z_ref)

    imp_ref[...] += jnp.sum(probs, axis=0, keepdims=True)
    cnt_ref[...] += jnp.sum(onehot, axis=0, keepdims=True)
    z_ref[...] += jnp.sum(lse * lse, axis=0, keepdims=True)[:, :1]

    @pl.when(t == n_blocks - 1)
    def _():
        z_ref[...] = z_ref[...] / jnp.float32(n_tok)
        imp = imp_ref[...] / jnp.float32(n_tok)
        fi = cnt_ref[...] / jnp.float32(n_tok)
        aux_ref[...] = jnp.float32(E) * jnp.sum(imp * fi, axis=1,
                                                keepdims=True)


def _dyngather(v, idx):
    return jax.lax.gather(
        v, idx[:, None],
        jax.lax.GatherDimensionNumbers(offset_dims=(), collapsed_slice_dims=(0,),
                                       start_index_map=(0,)),
        (1,), mode=jax.lax.GatherScatterMode.PROMISE_IN_BOUNDS)


def _dispatch_body(eid_hbm, ch_hbm, flat_hbm, w_hbm, xs_hbm, wsrow_hbm,
                   inv_hbm, emap_hbm, act_hbm, eid_v, ch_v, pos_v, posr_v,
                   tokr_v, rows_v, rows2_v, emap_v, act_v, w_v, wsr_v,
                   posr2_v, gsem0, gsem1, ssem0, ssem1, wsem):
    wid = lax.axis_index("s") * 2 + lax.axis_index("c")
    base = wid * CHUNK

    pltpu.sync_copy(eid_hbm.at[pl.ds(base, CHUNK)], eid_v)
    pltpu.sync_copy(w_hbm.at[pl.ds(base, CHUNK)], w_v)
    pltpu.sync_copy(ch_hbm, ch_v)

    lane = lax.iota(jnp.int32, 16)
    zero16 = jnp.zeros((16,), jnp.int32)
    total = zero16
    pre = zero16
    for t in range(N_CHUNKS):
        row = ch_v[t, pl.ds(0, 16)]
        keep = jnp.full((16,), t, jnp.int32) < wid
        total = total + row
        pre = pre + jnp.where(keep, row, zero16)

    starts = []
    s = jnp.int32(0)
    for e in range(E):
        starts.append(s)
        te = jnp.sum(jnp.where(lane == e, total, zero16))
        s = (s + te + jnp.int32(ROW_BLK - 1)) & jnp.int32(-ROW_BLK)
    starts.append(s)

    @pl.when(wid == 0)
    def _():
        for k in range(NTILES_PAD // 16):
            rb = (lane + 16 * k) * ROW_BLK
            cnt = zero16
            for e in range(1, E + 1):
                cnt = cnt + jnp.where(rb >= starts[e], 1, 0).astype(jnp.int32)
            emap_v[pl.ds(16 * k, 16)] = jnp.minimum(cnt, E - 1)
            act_v[pl.ds(16 * k, 16)] = jnp.where(rb < starts[E], 1,
                                                 0).astype(jnp.int32)
        pltpu.sync_copy(emap_v, emap_hbm)
        pltpu.sync_copy(act_v, act_hbm)

    run = []
    for e in range(E):
        pe = jnp.sum(jnp.where(lane == e, pre, zero16))
        run.append(starts[e] + pe)

    for j in range(CHUNK // 16):
        v = eid_v[pl.ds(16 * j, 16)]
        pos = zero16
        for e in range(E):
            m = v == e
            mi = m.astype(jnp.int32)
            c = plsc.cumsum(mi)
            pos = pos + jnp.where(m, run[e] + c - 1, zero16)
            run[e] = run[e] + jnp.sum(mi)
        pos_v[pl.ds(16 * j, 16)] = pos

    pltpu.sync_copy(pos_v, inv_hbm.at[pl.ds(base, CHUNK)])

    for j in range(CHUNK // 16):
        wv = w_v[pl.ds(16 * j, 16)]
        for m in range(16):
            wsr_v[16 * j + m, pl.ds(0, 16)] = _dyngather(
                wv, jnp.full((16,), m, jnp.int32))
    for b2 in range(2):
        for m in range(8):
            posr2_v[b2, pl.ds(16 * m, 16)] = pos_v[pl.ds(128 * b2 + 16 * m,
                                                         16)]
    pltpu.async_copy(wsr_v.at[pl.ds(0, 128), :],
                     wsrow_hbm.at[posr2_v.at[0]], wsem)
    pltpu.async_copy(wsr_v.at[pl.ds(128, 128), :],
                     wsrow_hbm.at[posr2_v.at[1]], wsem)

    for c in range(8):
        for m in range(2):
            idxv = jnp.full((16,), base + 32 * c + 16 * m, jnp.int32) + lane
            tokr_v[c, pl.ds(16 * m, 16)] = idxv >> 1
            posr_v[c, pl.ds(16 * m, 16)] = pos_v[pl.ds(32 * c + 16 * m, 16)]

    bufs = (rows_v, rows2_v)
    gsems = (gsem0, gsem1)
    ssems = (ssem0, ssem1)
    pltpu.async_copy(flat_hbm.at[tokr_v.at[0]], bufs[0], gsems[0])
    for c in range(8):
        b = c % 2
        o = 1 - b
        pltpu.make_async_copy(flat_hbm.at[pl.ds(0, 32)], bufs[b],
                              gsems[b]).wait()
        if c >= 1:
            pltpu.make_async_copy(bufs[o], xs_hbm.at[pl.ds(0, 32)],
                                  ssems[o]).wait()
        if c + 1 < 8:
            pltpu.async_copy(flat_hbm.at[tokr_v.at[c + 1]], bufs[o], gsems[o])
        pltpu.async_copy(bufs[b], xs_hbm.at[posr_v.at[c]], ssems[b])
    pltpu.make_async_copy(bufs[1], xs_hbm.at[pl.ds(0, 32)], ssems[1]).wait()
    pltpu.make_async_copy(wsr_v.at[pl.ds(0, 128), :],
                          wsrow_hbm.at[pl.ds(0, 128)], wsem).wait()
    pltpu.make_async_copy(wsr_v.at[pl.ds(0, 128), :],
                          wsrow_hbm.at[pl.ds(0, 128)], wsem).wait()


def _gffn_body(emap_ref, act_ref, xs_ref, ws_ref, w1_ref, b1_ref, w2_ref,
               b2_ref, ys_ref):
    r = pl.program_id(0)

    @pl.when(act_ref[r] > 0)
    def _():
        x = xs_ref[...]
        h = jax.lax.dot_general(x, w1_ref[0], (((1,), (0,)), ((), ())),
                                preferred_element_type=jnp.float32)
        h = jax.nn.gelu(h + b1_ref[0])
        y = jax.lax.dot_general(h, w2_ref[0], (((1,), (0,)), ((), ())),
                                preferred_element_type=jnp.float32)
        ys_ref[...] = (y + b2_ref[0]) * ws_ref[...][:, :1]


def _combine_body(ys_hbm, inv_hbm, out_hbm, inv_v, rows0_v, rows1_v, out0_v,
                  out1_v, gsem0, gsem1, osem0, osem1):
    wid = lax.axis_index("s") * 2 + lax.axis_index("c")
    base = wid * CHUNK
    tbase = wid * (CHUNK // 2)
    pltpu.sync_copy(inv_hbm.at[pl.ds(base, CHUNK)], inv_v)

    n_chunks = CHUNK // 16
    bufs = (rows0_v, rows1_v)
    gsems = (gsem0, gsem1)
    out_vs = (out0_v, out1_v)
    osems = (osem0, osem1)
    pltpu.async_copy(ys_hbm.at[inv_v[pl.ds(0, 16)]], bufs[0], gsems[0])
    pltpu.async_copy(ys_hbm.at[inv_v[pl.ds(16, 16)]], bufs[1], gsems[1])

    def chunk_body(g, carry):
        for b in range(2):
            c = 2 * g + b
            buf = bufs[b]
            pltpu.make_async_copy(ys_hbm.at[pl.ds(0, 16)], buf,
                                  gsems[b]).wait()
            ov = out_vs[b]

            @pl.when(c >= 2)
            def _():
                pltpu.make_async_copy(ov, out_hbm.at[pl.ds(0, 8)],
                                      osems[b]).wait()

            for i in range(8):
                for l in range(DIM // 16):
                    r0 = buf[2 * i, pl.ds(16 * l, 16)]
                    r1 = buf[2 * i + 1, pl.ds(16 * l, 16)]
                    ov[i, pl.ds(16 * l, 16)] = r0 + r1

            @pl.when(c + 2 < n_chunks)
            def _():
                nxt = inv_v[pl.ds(16 * (c + 2), 16)]
                pltpu.async_copy(ys_hbm.at[nxt], buf, gsems[b])

            pltpu.async_copy(ov, out_hbm.at[pl.ds(tbase + 8 * c, 8)],
                             osems[b])
        return carry

    lax.fori_loop(0, n_chunks // 2, chunk_body, 0)
    pltpu.make_async_copy(out_vs[0], out_hbm.at[pl.ds(0, 8)], osems[0]).wait()
    pltpu.make_async_copy(out_vs[1], out_hbm.at[pl.ds(0, 8)], osems[1]).wait()


def kernel(x, Wr, br, W1, b1, W2, b2):
    batch, n_tokens, dim = x.shape
    total = batch * n_tokens
    flat = x.reshape(total, dim)
    n_blocks = total // TOK_BLK

    wr_pad = jnp.zeros((LANES, dim), jnp.float32).at[:E].set(Wr)
    br_pad = jnp.zeros((1, LANES), jnp.float32).at[0, :E].set(br)

    router = pl.pallas_call(
        functools.partial(_router_body, n_tok=total, n_blocks=n_blocks),
        grid=(n_blocks,),
        in_specs=[
            pl.BlockSpec((TOK_BLK, dim), lambda t: (t, 0)),
            pl.BlockSpec((LANES, dim), lambda t: (0, 0)),
            pl.BlockSpec((1, LANES), lambda t: (0, 0)),
        ],
        out_specs=[
            pl.BlockSpec((TOK_BLK, 2), lambda t: (t, 0)),
            pl.BlockSpec((TOK_BLK, 2), lambda t: (t, 0)),
            pl.BlockSpec((1, 4, LANES), lambda t: (t, 0, 0)),
            pl.BlockSpec((1, 1), lambda t: (0, 0)),
            pl.BlockSpec((1, 1), lambda t: (0, 0)),
        ],
        out_shape=[
            jax.ShapeDtypeStruct((total, 2), jnp.int32),
            jax.ShapeDtypeStruct((total, 2), jnp.float32),
            jax.ShapeDtypeStruct((n_blocks, 4, LANES), jnp.int32),
            jax.ShapeDtypeStruct((1, 1), jnp.float32),
            jax.ShapeDtypeStruct((1, 1), jnp.float32),
        ],
        scratch_shapes=[
            pltpu.VMEM((1, LANES), jnp.float32),
            pltpu.VMEM((1, LANES), jnp.float32),
        ],
    )
    ti, tw, ch, z_loss, aux_loss = router(flat, wr_pad, br_pad)

    eid = ti.reshape(NSLOT)
    wflat = tw.reshape(NSLOT)
    ch2 = ch.reshape(N_CHUNKS, LANES)

    mesh = plsc.VectorSubcoreMesh(core_axis_name="c", subcore_axis_name="s",
                                  num_cores=2)
    dispatch = functools.partial(
        pl.kernel,
        out_type=[
            jax.ShapeDtypeStruct((NPAD, DIM), jnp.float32),
            jax.ShapeDtypeStruct((NPAD, 128), jnp.float32),
            jax.ShapeDtypeStruct((NSLOT,), jnp.int32),
            jax.ShapeDtypeStruct((NTILES_PAD,), jnp.int32),
            jax.ShapeDtypeStruct((NTILES_PAD,), jnp.int32),
        ],
        mesh=mesh,
        compiler_params=pltpu.CompilerParams(needs_layout_passes=False),
        scratch_types=[
            pltpu.VMEM((CHUNK,), jnp.int32),
            pltpu.VMEM((N_CHUNKS, LANES), jnp.int32),
            pltpu.VMEM((CHUNK,), jnp.int32),
            pltpu.VMEM((8, 32), jnp.int32),
            pltpu.VMEM((8, 32), jnp.int32),
            pltpu.VMEM((32, DIM), jnp.float32),
            pltpu.VMEM((32, DIM), jnp.float32),
            pltpu.VMEM((NTILES_PAD,), jnp.int32),
            pltpu.VMEM((NTILES_PAD,), jnp.int32),
            pltpu.VMEM((CHUNK,), jnp.float32),
            pltpu.VMEM((CHUNK, 128), jnp.float32),
            pltpu.VMEM((2, 128), jnp.int32),
            pltpu.SemaphoreType.DMA,
            pltpu.SemaphoreType.DMA,
            pltpu.SemaphoreType.DMA,
            pltpu.SemaphoreType.DMA,
            pltpu.SemaphoreType.DMA,
        ],
    )(_dispatch_body)
    xs, wsrow, inv, emap, act = dispatch(eid, ch2, flat, wflat)

    b1r = b1.reshape(E, 1, FF)
    b2r = b2.reshape(E, 1, DIM)
    grid_spec = pltpu.PrefetchScalarGridSpec(
        num_scalar_prefetch=2,
        grid=(NTILES,),
        in_specs=[
            pl.BlockSpec((ROW_BLK, DIM), lambda r, em, ac: (r, 0)),
            pl.BlockSpec((ROW_BLK, 128), lambda r, em, ac: (r, 0)),
            pl.BlockSpec((1, DIM, FF), lambda r, em, ac: (em[r], 0, 0)),
            pl.BlockSpec((1, 1, FF), lambda r, em, ac: (em[r], 0, 0)),
            pl.BlockSpec((1, FF, DIM), lambda r, em, ac: (em[r], 0, 0)),
            pl.BlockSpec((1, 1, DIM), lambda r, em, ac: (em[r], 0, 0)),
        ],
        out_specs=pl.BlockSpec((ROW_BLK, DIM), lambda r, em, ac: (r, 0)),
    )
    ys = pl.pallas_call(
        _gffn_body,
        grid_spec=grid_spec,
        out_shape=jax.ShapeDtypeStruct((NPAD, DIM), jnp.float32),
    )(emap, act, xs, wsrow, W1, b1r, W2, b2r)

    combine = functools.partial(
        pl.kernel,
        out_type=jax.ShapeDtypeStruct((TOTAL, DIM), jnp.float32),
        mesh=mesh,
        compiler_params=pltpu.CompilerParams(needs_layout_passes=False),
        scratch_types=[
            pltpu.VMEM((CHUNK,), jnp.int32),
            pltpu.VMEM((16, DIM), jnp.float32),
            pltpu.VMEM((16, DIM), jnp.float32),
            pltpu.VMEM((8, DIM), jnp.float32),
            pltpu.VMEM((8, DIM), jnp.float32),
            pltpu.SemaphoreType.DMA,
            pltpu.SemaphoreType.DMA,
            pltpu.SemaphoreType.DMA,
            pltpu.SemaphoreType.DMA,
        ],
    )(_combine_body)
    out = combine(ys, inv)

    return (out.reshape(batch, n_tokens, dim), z_loss.reshape(()),
            aux_loss.reshape(()))


# speedup vs baseline: 1.0665x; 1.0665x over previous
import functools

import jax
import jax.numpy as jnp
from jax import lax
from jax.experimental import pallas as pl
from jax.experimental.pallas import tpu as pltpu
from jax.experimental.pallas import tpu_sc as plsc

E = 8
DIM = 1024
FF = 2048
TOK_BLK = 512
LANES = 128
ROW_BLK = 256
TOTAL = 4096
NSLOT = 2 * TOTAL
NPAD = NSLOT + E * ROW_BLK
NTILES = NPAD // ROW_BLK
NTILES_PAD = 48
NW = 32
CHUNK = NSLOT // NW
N_CHUNKS = TOTAL // (TOK_BLK // 4)


def _router_body(x_ref, wr_ref, br_ref, ti_ref, tw_ref, ch_ref, z_ref,
                 aux_ref, imp_ref, cnt_ref, *, n_tok, n_blocks):
    t = pl.program_id(0)
    x = x_ref[...]
    wr = wr_ref[...]
    logits = jax.lax.dot_general(x, wr, (((1,), (1,)), ((), ())),
                                 preferred_element_type=jnp.float32)
    logits = logits + br_ref[...]
    lane = jax.lax.broadcasted_iota(jnp.int32, logits.shape, 1)
    valid = lane < E
    logits = jnp.where(valid, logits, jnp.float32(-1e30))

    m = jnp.max(logits, axis=1, keepdims=True)
    el = jnp.where(valid, jnp.exp(logits - m), 0.0)
    s = jnp.sum(el, axis=1, keepdims=True)
    probs = el / s
    lse = m + jnp.log(s)

    big = jnp.int32(LANES)
    p1 = jnp.max(probs, axis=1, keepdims=True)
    i1 = jnp.min(jnp.where(probs == p1, lane, big), axis=1, keepdims=True)
    probs2 = jnp.where(lane == i1, -1.0, probs)
    p2 = jnp.max(probs2, axis=1, keepdims=True)
    i2 = jnp.min(jnp.where(probs2 == p2, lane, big), axis=1, keepdims=True)

    denom = p1 + p2 + 1e-12
    ti_ref[...] = jnp.concatenate([i1, i2], axis=1)
    tw_ref[...] = jnp.concatenate([p1 / denom, p2 / denom], axis=1)

    onehot = (jnp.where(lane == i1, 1.0, 0.0) + jnp.where(lane == i2, 1.0, 0.0))
    sub = jnp.sum(onehot.reshape(4, TOK_BLK // 4, LANES), axis=1)
    ch_ref[...] = sub.astype(jnp.int32).reshape(1, 4, LANES)

    @pl.when(t == 0)
    def _():
        imp_ref[...] = jnp.zeros_like(imp_ref)
        cnt_ref[...] = jnp.zeros_like(cnt_ref)
        z_ref[...] = jnp.zeros_like(z_ref)

    imp_ref[...] += jnp.sum(probs, axis=0, keepdims=True)
    cnt_ref[...] += jnp.sum(onehot, axis=0, keepdims=True)
    z_ref[...] += jnp.sum(lse * lse, axis=0, keepdims=True)[:, :1]

    @pl.when(t == n_blocks - 1)
    def _():
        z_ref[...] = z_ref[...] / jnp.float32(n_tok)
        imp = imp_ref[...] / jnp.float32(n_tok)
        fi = cnt_ref[...] / jnp.float32(n_tok)
        aux_ref[...] = jnp.float32(E) * jnp.sum(imp * fi, axis=1,
                                                keepdims=True)


def _dyngather(v, idx):
    return jax.lax.gather(
        v, idx[:, None],
        jax.lax.GatherDimensionNumbers(offset_dims=(), collapsed_slice_dims=(0,),
                                       start_index_map=(0,)),
        (1,), mode=jax.lax.GatherScatterMode.PROMISE_IN_BOUNDS)


def _dispatch_body(eid_hbm, ch_hbm, flat_hbm, w_hbm, xs_hbm, wsrow_hbm,
                   inv_hbm, emap_hbm, act_hbm, eid_v, ch_v, pos_v, posr_v,
                   tokr_v, rows_v, rows2_v, emap_v, act_v, w_v, wsr_v,
                   posr2_v, gsem0, gsem1, ssem0, ssem1, wsem):
    wid = lax.axis_index("s") * 2 + lax.axis_index("c")
    base = wid * CHUNK

    pltpu.sync_copy(eid_hbm.at[pl.ds(base, CHUNK)], eid_v)
    pltpu.sync_copy(w_hbm.at[pl.ds(base, CHUNK)], w_v)
    pltpu.sync_copy(ch_hbm, ch_v)

    lane = lax.iota(jnp.int32, 16)
    zero16 = jnp.zeros((16,), jnp.int32)
    total = zero16
    pre = zero16
    for t in range(N_CHUNKS):
        row = ch_v[t, pl.ds(0, 16)]
        keep = jnp.full((16,), t, jnp.int32) < wid
        total = total + row
        pre = pre + jnp.where(keep, row, zero16)

    starts = []
    s = jnp.int32(0)
    for e in range(E):
        starts.append(s)
        te = jnp.sum(jnp.where(lane == e, total, zero16))
        s = (s + te + jnp.int32(ROW_BLK - 1)) & jnp.int32(-ROW_BLK)
    starts.append(s)

    @pl.when(wid == 0)
    def _():
        for k in range(NTILES_PAD // 16):
            rb = (lane + 16 * k) * ROW_BLK
            cnt = zero16
            for e in range(1, E + 1):
                cnt = cnt + jnp.where(rb >= starts[e], 1, 0).astype(jnp.int32)
            emap_v[pl.ds(16 * k, 16)] = jnp.minimum(cnt, E - 1)
            act_v[pl.ds(16 * k, 16)] = jnp.where(rb < starts[E], 1,
                                                 0).astype(jnp.int32)
        pltpu.sync_copy(emap_v, emap_hbm)
        pltpu.sync_copy(act_v, act_hbm)

    run = []
    for e in range(E):
        pe = jnp.sum(jnp.where(lane == e, pre, zero16))
        run.append(starts[e] + pe)

    for j in range(CHUNK // 16):
        v = eid_v[pl.ds(16 * j, 16)]
        pos = zero16
        for e in range(E):
            m = v == e
            mi = m.astype(jnp.int32)
            c = plsc.cumsum(mi)
            pos = pos + jnp.where(m, run[e] + c - 1, zero16)
            run[e] = run[e] + jnp.sum(mi)
        pos_v[pl.ds(16 * j, 16)] = pos

    pltpu.sync_copy(pos_v, inv_hbm.at[pl.ds(base, CHUNK)])

    for j in range(CHUNK // 16):
        wv = w_v[pl.ds(16 * j, 16)]
        for m in range(16):
            wsr_v[16 * j + m, pl.ds(0, 16)] = _dyngather(
                wv, jnp.full((16,), m, jnp.int32))
    for b2 in range(2):
        for m in range(8):
            posr2_v[b2, pl.ds(16 * m, 16)] = pos_v[pl.ds(128 * b2 + 16 * m,
                                                         16)]
    pltpu.async_copy(wsr_v.at[pl.ds(0, 128), :],
                     wsrow_hbm.at[posr2_v.at[0]], wsem)
    pltpu.async_copy(wsr_v.at[pl.ds(128, 128), :],
                     wsrow_hbm.at[posr2_v.at[1]], wsem)

    for c in range(8):
        for m in range(2):
            idxv = jnp.full((16,), base + 32 * c + 16 * m, jnp.int32) + lane
            tokr_v[c, pl.ds(16 * m, 16)] = idxv >> 1
            posr_v[c, pl.ds(16 * m, 16)] = pos_v[pl.ds(32 * c + 16 * m, 16)]

    bufs = (rows_v, rows2_v)
    gsems = (gsem0, gsem1)
    ssems = (ssem0, ssem1)
    pltpu.async_copy(flat_hbm.at[tokr_v.at[0]], bufs[0], gsems[0])
    for c in range(8):
        b = c % 2
        o = 1 - b
        pltpu.make_async_copy(flat_hbm.at[pl.ds(0, 32)], bufs[b],
                              gsems[b]).wait()
        if c >= 1:
            pltpu.make_async_copy(bufs[o], xs_hbm.at[pl.ds(0, 32)],
                                  ssems[o]).wait()
        if c + 1 < 8:
            pltpu.async_copy(flat_hbm.at[tokr_v.at[c + 1]], bufs[o], gsems[o])
        pltpu.async_copy(bufs[b], xs_hbm.at[posr_v.at[c]], ssems[b])
    pltpu.make_async_copy(bufs[1], xs_hbm.at[pl.ds(0, 32)], ssems[1]).wait()
    pltpu.make_async_copy(wsr_v.at[pl.ds(0, 128), :],
                          wsrow_hbm.at[pl.ds(0, 128)], wsem).wait()
    pltpu.make_async_copy(wsr_v.at[pl.ds(0, 128), :],
                          wsrow_hbm.at[pl.ds(0, 128)], wsem).wait()


def _gffn_body(emap_ref, act_ref, xs_ref, ws_ref, w1_ref, b1_ref, w2_ref,
               b2_ref, ys_ref):
    r = pl.program_id(0)

    @pl.when(act_ref[r] > 0)
    def _():
        x = xs_ref[...]
        h = jax.lax.dot_general(x, w1_ref[0], (((1,), (0,)), ((), ())),
                                preferred_element_type=jnp.float32)
        h = jax.nn.gelu(h + b1_ref[0])
        y = jax.lax.dot_general(h, w2_ref[0], (((1,), (0,)), ((), ())),
                                preferred_element_type=jnp.float32)
        ys_ref[...] = (y + b2_ref[0]) * ws_ref[...][:, :1]


def _combine_body(ys_hbm, inv_hbm, out_hbm, inv_v, rows0_v, rows1_v, out0_v,
                  out1_v, gsem0, gsem1, osem0, osem1):
    wid = lax.axis_index("s") * 2 + lax.axis_index("c")
    base = wid * CHUNK
    tbase = wid * (CHUNK // 2)
    pltpu.sync_copy(inv_hbm.at[pl.ds(base, CHUNK)], inv_v)

    n_chunks = CHUNK // 16
    bufs = (rows0_v, rows1_v)
    gsems = (gsem0, gsem1)
    out_vs = (out0_v, out1_v)
    osems = (osem0, osem1)
    pltpu.async_copy(ys_hbm.at[inv_v[pl.ds(0, 16)]], bufs[0], gsems[0])
    pltpu.async_copy(ys_hbm.at[inv_v[pl.ds(16, 16)]], bufs[1], gsems[1])

    def chunk_body(g, carry):
        for b in range(2):
            c = 2 * g + b
            buf = bufs[b]
            pltpu.make_async_copy(ys_hbm.at[pl.ds(0, 16)], buf,
                                  gsems[b]).wait()
            ov = out_vs[b]

            @pl.when(c >= 2)
            def _():
                pltpu.make_async_copy(ov, out_hbm.at[pl.ds(0, 8)],
                                      osems[b]).wait()

            for i in range(8):
                for l in range(DIM // 16):
                    r0 = buf[2 * i, pl.ds(16 * l, 16)]
                    r1 = buf[2 * i + 1, pl.ds(16 * l, 16)]
                    ov[i, pl.ds(16 * l, 16)] = r0 + r1

            @pl.when(c + 2 < n_chunks)
            def _():
                nxt = inv_v[pl.ds(16 * (c + 2), 16)]
                pltpu.async_copy(ys_hbm.at[nxt], buf, gsems[b])

            pltpu.async_copy(ov, out_hbm.at[pl.ds(tbase + 8 * c, 8)],
                             osems[b])
        return carry

    lax.fori_loop(0, n_chunks // 2, chunk_body, 0)
    pltpu.make_async_copy(out_vs[0], out_hbm.at[pl.ds(0, 8)], osems[0]).wait()
    pltpu.make_async_copy(out_vs[1], out_hbm.at[pl.ds(0, 8)], osems[1]).wait()


def kernel(x, Wr, br, W1, b1, W2, b2):
    batch, n_tokens, dim = x.shape
    total = batch * n_tokens
    flat = x.reshape(total, dim)
    n_blocks = total // TOK_BLK

    wr_pad = jnp.zeros((LANES, dim), jnp.float32).at[:E].set(Wr)
    br_pad = jnp.zeros((1, LANES), jnp.float32).at[0, :E].set(br)

    router = pl.pallas_call(
        functools.partial(_router_body, n_tok=total, n_blocks=n_blocks),
        grid=(n_blocks,),
        in_specs=[
            pl.BlockSpec((TOK_BLK, dim), lambda t: (t, 0)),
            pl.BlockSpec((LANES, dim), lambda t: (0, 0)),
            pl.BlockSpec((1, LANES), lambda t: (0, 0)),
        ],
        out_specs=[
            pl.BlockSpec((TOK_BLK, 2), lambda t: (t, 0)),
            pl.BlockSpec((TOK_BLK, 2), lambda t: (t, 0)),
            pl.BlockSpec((1, 4, LANES), lambda t: (t, 0, 0)),
            pl.BlockSpec((1, 1), lambda t: (0, 0)),
            pl.BlockSpec((1, 1), lambda t: (0, 0)),
        ],
        out_shape=[
            jax.ShapeDtypeStruct((total, 2), jnp.int32),
            jax.ShapeDtypeStruct((total, 2), jnp.float32),
            jax.ShapeDtypeStruct((n_blocks, 4, LANES), jnp.int32),
            jax.ShapeDtypeStruct((1, 1), jnp.float32),
            jax.ShapeDtypeStruct((1, 1), jnp.float32),
        ],
        scratch_shapes=[
            pltpu.VMEM((1, LANES), jnp.float32),
            pltpu.VMEM((1, LANES), jnp.float32),
        ],
    )
    ti, tw, ch, z_loss, aux_loss = router(flat, wr_pad, br_pad)

    eid = ti.reshape(NSLOT)
    wflat = tw.reshape(NSLOT)
    ch2 = ch.reshape(N_CHUNKS, LANES)

    mesh = plsc.VectorSubcoreMesh(core_axis_name="c", subcore_axis_name="s",
                                  num_cores=2)
    dispatch = functools.partial(
        pl.kernel,
        out_type=[
            jax.ShapeDtypeStruct((NPAD, DIM), jnp.float32),
            jax.ShapeDtypeStruct((NPAD, 128), jnp.float32),
            jax.ShapeDtypeStruct((NSLOT,), jnp.int32),
            jax.ShapeDtypeStruct((NTILES_PAD,), jnp.int32),
            jax.ShapeDtypeStruct((NTILES_PAD,), jnp.int32),
        ],
        mesh=mesh,
        compiler_params=pltpu.CompilerParams(needs_layout_passes=False),
        scratch_types=[
            pltpu.VMEM((CHUNK,), jnp.int32),
            pltpu.VMEM((N_CHUNKS, LANES), jnp.int32),
            pltpu.VMEM((CHUNK,), jnp.int32),
            pltpu.VMEM((8, 32), jnp.int32),
            pltpu.VMEM((8, 32), jnp.int32),
            pltpu.VMEM((32, DIM), jnp.float32),
            pltpu.VMEM((32, DIM), jnp.float32),
            pltpu.VMEM((NTILES_PAD,), jnp.int32),
            pltpu.VMEM((NTILES_PAD,), jnp.int32),
            pltpu.VMEM((CHUNK,), jnp.float32),
            pltpu.VMEM((CHUNK, 128), jnp.float32),
            pltpu.VMEM((2, 128), jnp.int32),
            pltpu.SemaphoreType.DMA,
            pltpu.SemaphoreType.DMA,
            pltpu.SemaphoreType.DMA,
            pltpu.SemaphoreType.DMA,
            pltpu.SemaphoreType.DMA,
        ],
    )(_dispatch_body)
    xs, wsrow, inv, emap, act = dispatch(eid, ch2, flat, wflat)

    b1r = b1.reshape(E, 1, FF)
    b2r = b2.reshape(E, 1, DIM)
    grid_spec = pltpu.PrefetchScalarGridSpec(
        num_scalar_prefetch=2,
        grid=(NTILES,),
        in_specs=[
            pl.BlockSpec((ROW_BLK, DIM), lambda r, em, ac: (r, 0)),
            pl.BlockSpec((ROW_BLK, 128), lambda r, em, ac: (r, 0)),
            pl.BlockSpec((1, DIM, FF), lambda r, em, ac: (em[r], 0, 0)),
            pl.BlockSpec((1, 1, FF), lambda r, em, ac: (em[r], 0, 0)),
            pl.BlockSpec((1, FF, DIM), lambda r, em, ac: (em[r], 0, 0)),
            pl.BlockSpec((1, 1, DIM), lambda r, em, ac: (em[r], 0, 0)),
        ],
        out_specs=pl.BlockSpec((ROW_BLK, DIM), lambda r, em, ac: (r, 0)),
    )
    ys = pl.pallas_call(
        _gffn_body,
        grid_spec=grid_spec,
        out_shape=jax.ShapeDtypeStruct((NPAD, DIM), jnp.float32),
    )(emap, act, xs, wsrow, W1, b1r, W2, b2r)

    combine = functools.partial(
        pl.kernel,
        out_type=jax.ShapeDtypeStruct((TOTAL, DIM), jnp.float32),
        mesh=mesh,
        compiler_params=pltpu.CompilerParams(needs_layout_passes=False),
        scratch_types=[
            pltpu.VMEM((CHUNK,), jnp.int32),
            pltpu.VMEM((16, DIM), jnp.float32),
            pltpu.VMEM((16, DIM), jnp.float32),
            pltpu.VMEM((8, DIM), jnp.float32),
            pltpu.VMEM((8, DIM), jnp.float32),
            pltpu.SemaphoreType.DMA,
            pltpu.SemaphoreType.DMA,
            pltpu.SemaphoreType.DMA,
            pltpu.SemaphoreType.DMA,
        ],
    )(_combine_body)
    out = combine(ys, inv)

    return (out.reshape(batch, n_tokens, dim), z_loss.reshape(()),
            aux_loss.reshape(()))
